# baseline (device time: 138522 ns/iter reference)
import jax
import jax.numpy as jnp
from jax import lax
from jax.experimental import pallas as pl
from jax.experimental.pallas import tpu as pltpu

N_DEV = 16
M = 4096
N = 1024
HN = N // 2
PCH = 1024
QCH = 256
S1 = 4

NSEM = 30


def kernel(x):
    def body(
        x_ref,
        out_ref,
        p1s_f, p1r_f, p1s_b, p1r_b,
        res1_f, res1_b,
        p2s_f, p2r_f, p2s_b, p2r_b,
        red_f, red_b,
        resp_f, resp_b,
        p3_f, p3_b,
        p4_f, p4_b,
        ssem_f, rsem_f, ssem_b, rsem_b,
    ):
        my = lax.axis_index("i")
        z = my // 4
        j = lax.rem(my, 4)
        succ_p = 4 * z + lax.rem(j + 1, 4)
        pred_p = 4 * z + lax.rem(j + 3, 4)
        succ_c = 4 * lax.rem(z + 1, 4) + j
        pred_c = 4 * lax.rem(z + 3, 4) + j

        def rdma(src, dst, sem_i, fwd, column):
            if fwd:
                tgt, ssem, rsem = (succ_c if column else succ_p), ssem_f, rsem_f
            else:
                tgt, ssem, rsem = (pred_c if column else pred_p), ssem_b, rsem_b
            return pltpu.make_async_remote_copy(
                src_ref=src,
                dst_ref=dst,
                send_sem=ssem.at[sem_i],
                recv_sem=rsem.at[sem_i],
                device_id=(tgt,),
                device_id_type=pl.DeviceIdType.MESH,
            )

        def p1_rdma(h, s, fwd):
            a, b = (p1s_f, p1r_f) if fwd else (p1s_b, p1r_b)
            return rdma(a.at[h, s], b.at[h, s], h * 4 + s, fwd, False)

        def p2_rdma(h, fwd):
            a, b = (p2s_f, p2r_f) if fwd else (p2s_b, p2r_b)
            return rdma(a.at[h], b.at[h], 12 + h, fwd, True)

        def p3_rdma(h, fwd):
            p3, red = (p3_f, red_f) if fwd else (p3_b, red_b)
            src = red if h == 0 else p3.at[h - 1]
            return rdma(src, p3.at[h], 15 + h, fwd, True)

        def p4_rdma(h, s, fwd):
            p4, resp = (p4_f, resp_f) if fwd else (p4_b, resp_b)
            src = resp.at[s] if h == 0 else p4.at[h - 1, s]
            return rdma(src, p4.at[h, s], 18 + h * 4 + s, fwd, False)

        def xsub(c, s, fwd):
            cols = slice(0, HN) if fwd else slice(HN, N)
            return x_ref[pl.ds(c * PCH + s * QCH, QCH), cols].astype(
                jnp.bfloat16
            )

        barrier_sem = pltpu.get_barrier_semaphore()
        for nbr in (succ_p, pred_p, succ_c, pred_c):
            pl.semaphore_signal(
                barrier_sem, inc=1, device_id=(nbr,),
                device_id_type=pl.DeviceIdType.MESH,
            )
        pl.semaphore_wait(barrier_sem, 4)

        for s in range(S1):
            p1s_f[0, s, :, :] = xsub(j, s, True)
            p1_rdma(0, s, True).start()
            p1s_b[0, s, :, :] = xsub(j, s, False)
            p1_rdma(0, s, False).start()
        for h in range(3):
            cf = lax.rem(j - (h + 1) + 8, 4)
            cb = lax.rem(j + h + 1, 4)
            for s in range(S1):
                p1_rdma(h, s, True).wait_recv()
                if h < 2:
                    p1s_f[h + 1, s, :, :] = p1r_f[h, s] + xsub(cf, s, True)
                    p1_rdma(h + 1, s, True).start()
                else:
                    res1_f[s, :, :] = p1r_f[h, s] + xsub(cf, s, True)
                p1_rdma(h, s, False).wait_recv()
                if h < 2:
                    p1s_b[h + 1, s, :, :] = p1r_b[h, s] + xsub(cb, s, False)
                    p1_rdma(h + 1, s, False).start()
                else:
                    res1_b[s, :, :] = p1r_b[h, s] + xsub(cb, s, False)

        P = lax.rem(j + 1, 4)
        Pb = lax.rem(j + 3, 4)

        p2s_f[0, :, :] = res1_f[z]
        p2_rdma(0, True).start()
        p2s_b[0, :, :] = res1_b[z]
        p2_rdma(0, False).start()
        for h in range(3):
            qf = lax.rem(z - (h + 1) + 8, 4)
            qb = lax.rem(z + h + 1, 4)
            p2_rdma(h, True).wait_recv()
            if h < 2:
                p2s_f[h + 1, :, :] = p2r_f[h] + res1_f[qf]
                p2_rdma(h + 1, True).start()
            else:
                red_f[:, :] = p2r_f[h] + res1_f[qf]
                p3_rdma(0, True).start()
            p2_rdma(h, False).wait_recv()
            if h < 2:
                p2s_b[h + 1, :, :] = p2r_b[h] + res1_b[qb]
                p2_rdma(h + 1, False).start()
            else:
                red_b[:, :] = p2r_b[h] + res1_b[qb]
                p3_rdma(0, False).start()

        Q = lax.rem(z + 1, 4)
        Qb = lax.rem(z + 3, 4)
        resp_f[Q, :, :] = red_f[:, :]
        out_ref[pl.ds(P * PCH + Q * QCH, QCH), 0:HN] = red_f[:, :].astype(
            jnp.float32
        )
        resp_b[Qb, :, :] = red_b[:, :]
        out_ref[pl.ds(Pb * PCH + Qb * QCH, QCH), HN:N] = red_b[:, :].astype(
            jnp.float32
        )

        for h in range(3):
            qf = lax.rem(z - h + 8, 4)
            qb = lax.rem(z + h, 4)
            p3_rdma(h, True).wait_recv()
            if h < 2:
                p3_rdma(h + 1, True).start()
            resp_f[qf, :, :] = p3_f[h]
            out_ref[pl.ds(P * PCH + qf * QCH, QCH), 0:HN] = p3_f[h].astype(
                jnp.float32
            )
            p3_rdma(h, False).wait_recv()
            if h < 2:
                p3_rdma(h + 1, False).start()
            resp_b[qb, :, :] = p3_b[h]
            out_ref[pl.ds(Pb * PCH + qb * QCH, QCH), HN:N] = p3_b[h].astype(
                jnp.float32
            )

        for s in range(S1):
            p4_rdma(0, s, True).start()
            p4_rdma(0, s, False).start()
        for h in range(3):
            chf = lax.rem(j - h + 8, 4)
            chb = lax.rem(j + h, 4)
            for s in range(S1):
                p4_rdma(h, s, True).wait_recv()
                if h < 2:
                    p4_rdma(h + 1, s, True).start()
                p4_rdma(h, s, False).wait_recv()
                if h < 2:
                    p4_rdma(h + 1, s, False).start()
            out_ref[pl.ds(chf * PCH, PCH), 0:HN] = (
                p4_f[h].astype(jnp.float32).reshape(PCH, HN)
            )
            out_ref[pl.ds(chb * PCH, PCH), HN:N] = (
                p4_b[h].astype(jnp.float32).reshape(PCH, HN)
            )

        for h in range(3):
            for s in range(S1):
                p1_rdma(h, s, True).wait_send()
                p1_rdma(h, s, False).wait_send()
                p4_rdma(h, s, True).wait_send()
                p4_rdma(h, s, False).wait_send()
            p2_rdma(h, True).wait_send()
            p2_rdma(h, False).wait_send()
            p3_rdma(h, True).wait_send()
            p3_rdma(h, False).wait_send()

    bf = jnp.bfloat16
    return pl.pallas_call(
        body,
        out_shape=jax.ShapeDtypeStruct((M, N), jnp.float32),
        in_specs=[pl.BlockSpec(memory_space=pltpu.VMEM)],
        out_specs=pl.BlockSpec(memory_space=pltpu.VMEM),
        scratch_shapes=[
            pltpu.VMEM((3, S1, QCH, HN), bf),
            pltpu.VMEM((3, S1, QCH, HN), bf),
            pltpu.VMEM((3, S1, QCH, HN), bf),
            pltpu.VMEM((3, S1, QCH, HN), bf),
            pltpu.VMEM((4, QCH, HN), bf),
            pltpu.VMEM((4, QCH, HN), bf),
            pltpu.VMEM((3, QCH, HN), bf),
            pltpu.VMEM((3, QCH, HN), bf),
            pltpu.VMEM((3, QCH, HN), bf),
            pltpu.VMEM((3, QCH, HN), bf),
            pltpu.VMEM((QCH, HN), bf),
            pltpu.VMEM((QCH, HN), bf),
            pltpu.VMEM((4, QCH, HN), bf),
            pltpu.VMEM((4, QCH, HN), bf),
            pltpu.VMEM((3, QCH, HN), bf),
            pltpu.VMEM((3, QCH, HN), bf),
            pltpu.VMEM((3, S1, QCH, HN), bf),
            pltpu.VMEM((3, S1, QCH, HN), bf),
            pltpu.SemaphoreType.DMA((NSEM,)),
            pltpu.SemaphoreType.DMA((NSEM,)),
            pltpu.SemaphoreType.DMA((NSEM,)),
            pltpu.SemaphoreType.DMA((NSEM,)),
        ],
        compiler_params=pltpu.CompilerParams(
            vmem_limit_bytes=100 * 1024 * 1024,
            collective_id=0,
        ),
    )(x)


# device time: 119794 ns/iter; 1.1563x vs baseline; 1.1563x over previous
import jax
import jax.numpy as jnp
from jax import lax
from jax.experimental import pallas as pl
from jax.experimental.pallas import tpu as pltpu

N_DEV = 16
M = 4096
N = 1024
HN = N // 2
QW = 256
PCH = 1024
QCH = 256
S1 = 2
SR1 = PCH // S1

NSEM = 18


def kernel(x):
    def body(
        x_ref,
        out_ref,
        p1s_f, p1r_f, p1s_b, p1r_b,
        res1_f, res1_b,
        p2s_f, p2r_f, p2s_b, p2r_b,
        red_f, red_b,
        resp_f, resp_b,
        p3_f, p3_b,
        p4_f, p4_b,
        ssem_f, rsem_f, ssem_b, rsem_b,
    ):
        my = lax.axis_index("i")
        z = my // 4
        j = lax.rem(my, 4)
        succ_p = 4 * z + lax.rem(j + 1, 4)
        pred_p = 4 * z + lax.rem(j + 3, 4)
        succ_c = 4 * lax.rem(z + 1, 4) + j
        pred_c = 4 * lax.rem(z + 3, 4) + j
        P = lax.rem(j + 1, 4)
        Pb = lax.rem(j + 3, 4)
        Q = lax.rem(z + 1, 4)
        Qb = lax.rem(z + 3, 4)

        def rdma(src, dst, sl, sem_i, fwd, column):
            if fwd:
                tgt, ssem, rsem = (succ_c if column else succ_p), ssem_f, rsem_f
            else:
                tgt, ssem, rsem = (pred_c if column else pred_p), ssem_b, rsem_b
            return pltpu.make_async_remote_copy(
                src_ref=src,
                dst_ref=dst,
                send_sem=ssem.at[sl, sem_i],
                recv_sem=rsem.at[sl, sem_i],
                device_id=(tgt,),
                device_id_type=pl.DeviceIdType.MESH,
            )

        def p1_rdma(sl, h, s, fwd):
            a, b = (p1s_f, p1r_f) if fwd else (p1s_b, p1r_b)
            return rdma(a.at[sl, h, s], b.at[sl, h, s], sl, h * 2 + s, fwd, False)

        def p2_rdma(sl, h, fwd):
            a, b = (p2s_f, p2r_f) if fwd else (p2s_b, p2r_b)
            return rdma(a.at[sl, h], b.at[sl, h], sl, 6 + h, fwd, True)

        def p3_rdma(sl, h, fwd):
            p3, red = (p3_f, red_f) if fwd else (p3_b, red_b)
            src = red.at[sl] if h == 0 else p3.at[sl, h - 1]
            return rdma(src, p3.at[sl, h], sl, 9 + h, fwd, True)

        def p4_rdma(sl, h, s, fwd):
            p4, resp = (p4_f, resp_f) if fwd else (p4_b, resp_b)
            if h == 0:
                src = resp.at[sl, 2 * s : 2 * s + 2]
            else:
                src = p4.at[sl, h - 1, s]
            return rdma(src, p4.at[sl, h, s], sl, 12 + h * 2 + s, fwd, False)

        def xsub(c, s, fwd, sl):
            c0 = (0 if fwd else HN) + sl * QW
            return x_ref[
                pl.ds(c * PCH + s * SR1, SR1), c0 : c0 + QW
            ].astype(jnp.bfloat16)

        def ocols(fwd, sl):
            c0 = (0 if fwd else HN) + sl * QW
            return slice(c0, c0 + QW)

        barrier_sem = pltpu.get_barrier_semaphore()
        for nbr in (succ_p, pred_p, succ_c, pred_c):
            pl.semaphore_signal(
                barrier_sem, inc=1, device_id=(nbr,),
                device_id_type=pl.DeviceIdType.MESH,
            )
        pl.semaphore_wait(barrier_sem, 4)

        def p1_seed(sl):
            for s in range(S1):
                p1s_f[sl, 0, s, :, :] = xsub(j, s, True, sl)
                p1_rdma(sl, 0, s, True).start()
                p1s_b[sl, 0, s, :, :] = xsub(j, s, False, sl)
                p1_rdma(sl, 0, s, False).start()

        def p1_hop(sl, h):
            cf = lax.rem(j - (h + 1) + 8, 4)
            cb = lax.rem(j + h + 1, 4)
            for s in range(S1):
                p1_rdma(sl, h, s, True).wait_recv()
                if h < 2:
                    p1s_f[sl, h + 1, s, :, :] = (
                        p1r_f[sl, h, s] + xsub(cf, s, True, sl)
                    )
                    p1_rdma(sl, h + 1, s, True).start()
                else:
                    res1_f[sl, 2 * s : 2 * s + 2, :, :] = (
                        p1r_f[sl, h, s] + xsub(cf, s, True, sl)
                    ).reshape(2, QCH, QW)
                p1_rdma(sl, h, s, False).wait_recv()
                if h < 2:
                    p1s_b[sl, h + 1, s, :, :] = (
                        p1r_b[sl, h, s] + xsub(cb, s, False, sl)
                    )
                    p1_rdma(sl, h + 1, s, False).start()
                else:
                    res1_b[sl, 2 * s : 2 * s + 2, :, :] = (
                        p1r_b[sl, h, s] + xsub(cb, s, False, sl)
                    ).reshape(2, QCH, QW)

        def p2_seed(sl):
            p2s_f[sl, 0, :, :] = res1_f[sl, z]
            p2_rdma(sl, 0, True).start()
            p2s_b[sl, 0, :, :] = res1_b[sl, z]
            p2_rdma(sl, 0, False).start()

        def p2_hop(sl, h):
            qf = lax.rem(z - (h + 1) + 8, 4)
            qb = lax.rem(z + h + 1, 4)
            p2_rdma(sl, h, True).wait_recv()
            if h < 2:
                p2s_f[sl, h + 1, :, :] = p2r_f[sl, h] + res1_f[sl, qf]
                p2_rdma(sl, h + 1, True).start()
            else:
                red_f[sl, :, :] = p2r_f[sl, h] + res1_f[sl, qf]
                p3_rdma(sl, 0, True).start()
            p2_rdma(sl, h, False).wait_recv()
            if h < 2:
                p2s_b[sl, h + 1, :, :] = p2r_b[sl, h] + res1_b[sl, qb]
                p2_rdma(sl, h + 1, False).start()
            else:
                red_b[sl, :, :] = p2r_b[sl, h] + res1_b[sl, qb]
                p3_rdma(sl, 0, False).start()
            if h == 2:
                resp_f[sl, Q, :, :] = red_f[sl]
                out_ref[pl.ds(P * PCH + Q * QCH, QCH), ocols(True, sl)] = (
                    red_f[sl].astype(jnp.float32)
                )
                resp_b[sl, Qb, :, :] = red_b[sl]
                out_ref[pl.ds(Pb * PCH + Qb * QCH, QCH), ocols(False, sl)] = (
                    red_b[sl].astype(jnp.float32)
                )

        def p3_hop(sl, h):
            qf = lax.rem(z - h + 8, 4)
            qb = lax.rem(z + h, 4)
            p3_rdma(sl, h, True).wait_recv()
            if h < 2:
                p3_rdma(sl, h + 1, True).start()
            resp_f[sl, qf, :, :] = p3_f[sl, h]
            out_ref[pl.ds(P * PCH + qf * QCH, QCH), ocols(True, sl)] = (
                p3_f[sl, h].astype(jnp.float32)
            )
            p3_rdma(sl, h, False).wait_recv()
            if h < 2:
                p3_rdma(sl, h + 1, False).start()
            resp_b[sl, qb, :, :] = p3_b[sl, h]
            out_ref[pl.ds(Pb * PCH + qb * QCH, QCH), ocols(False, sl)] = (
                p3_b[sl, h].astype(jnp.float32)
            )

        def p4_seed(sl):
            for s in range(S1):
                p4_rdma(sl, 0, s, True).start()
                p4_rdma(sl, 0, s, False).start()

        def p4_hop(sl, h):
            chf = lax.rem(j - h + 8, 4)
            chb = lax.rem(j + h, 4)
            for s in range(S1):
                p4_rdma(sl, h, s, True).wait_recv()
                if h < 2:
                    p4_rdma(sl, h + 1, s, True).start()
                p4_rdma(sl, h, s, False).wait_recv()
                if h < 2:
                    p4_rdma(sl, h + 1, s, False).start()
            out_ref[pl.ds(chf * PCH, PCH), ocols(True, sl)] = (
                p4_f[sl, h].astype(jnp.float32).reshape(PCH, QW)
            )
            out_ref[pl.ds(chb * PCH, PCH), ocols(False, sl)] = (
                p4_b[sl, h].astype(jnp.float32).reshape(PCH, QW)
            )

        p1_seed(0)
        for h in range(3):
            p1_hop(0, h)
        p2_seed(0)
        p1_seed(1)
        for h in range(3):
            p2_hop(0, h)
            p1_hop(1, h)
        p2_seed(1)
        for h in range(3):
            p3_hop(0, h)
            p2_hop(1, h)
        p4_seed(0)
        for h in range(3):
            p4_hop(0, h)
            p3_hop(1, h)
        p4_seed(1)
        for h in range(3):
            p4_hop(1, h)

        for sl in range(2):
            for h in range(3):
                for s in range(S1):
                    p1_rdma(sl, h, s, True).wait_send()
                    p1_rdma(sl, h, s, False).wait_send()
                    p4_rdma(sl, h, s, True).wait_send()
                    p4_rdma(sl, h, s, False).wait_send()
                p2_rdma(sl, h, True).wait_send()
                p2_rdma(sl, h, False).wait_send()
                p3_rdma(sl, h, True).wait_send()
                p3_rdma(sl, h, False).wait_send()

    bf = jnp.bfloat16
    return pl.pallas_call(
        body,
        out_shape=jax.ShapeDtypeStruct((M, N), jnp.float32),
        in_specs=[pl.BlockSpec(memory_space=pltpu.VMEM)],
        out_specs=pl.BlockSpec(memory_space=pltpu.VMEM),
        scratch_shapes=[
            pltpu.VMEM((2, 3, S1, SR1, QW), bf),
            pltpu.VMEM((2, 3, S1, SR1, QW), bf),
            pltpu.VMEM((2, 3, S1, SR1, QW), bf),
            pltpu.VMEM((2, 3, S1, SR1, QW), bf),
            pltpu.VMEM((2, 4, QCH, QW), bf),
            pltpu.VMEM((2, 4, QCH, QW), bf),
            pltpu.VMEM((2, 3, QCH, QW), bf),
            pltpu.VMEM((2, 3, QCH, QW), bf),
            pltpu.VMEM((2, 3, QCH, QW), bf),
            pltpu.VMEM((2, 3, QCH, QW), bf),
            pltpu.VMEM((2, QCH, QW), bf),
            pltpu.VMEM((2, QCH, QW), bf),
            pltpu.VMEM((2, 4, QCH, QW), bf),
            pltpu.VMEM((2, 4, QCH, QW), bf),
            pltpu.VMEM((2, 3, QCH, QW), bf),
            pltpu.VMEM((2, 3, QCH, QW), bf),
            pltpu.VMEM((2, 3, S1, 2, QCH, QW), bf),
            pltpu.VMEM((2, 3, S1, 2, QCH, QW), bf),
            pltpu.SemaphoreType.DMA((2, NSEM)),
            pltpu.SemaphoreType.DMA((2, NSEM)),
            pltpu.SemaphoreType.DMA((2, NSEM)),
            pltpu.SemaphoreType.DMA((2, NSEM)),
        ],
        compiler_params=pltpu.CompilerParams(
            vmem_limit_bytes=100 * 1024 * 1024,
            collective_id=0,
        ),
    )(x)
